# Initial kernel scaffold; baseline (speedup 1.0000x reference)
import jax
import jax.numpy as jnp
from jax.experimental import pallas as pl


def kernel(user_table, item_table, keyword_table, user_ids, item_ids, keyword_ids, query_sizes, negative_item_ids):
    raise NotImplementedError("write your pallas kernel here")


# speedup vs baseline: 1.5521x; 1.5521x over previous
import functools

import jax
import jax.numpy as jnp
from jax import lax
from jax.experimental import pallas as pl
from jax.experimental.pallas import tpu as pltpu
from jax.experimental.pallas import tpu_sc as plsc

D = 32
L_KW = 20
NNEG = 4
LANES = 16
NC, NS = 2, 16
NW = NC * NS
CH = 128


def _splat(v):
    return jnp.full((LANES,), v, jnp.int32)


def _fm_diffs_sc(utab, itab, ktab, uid, iid, kw_flat, qs, negid):
    B = uid.shape[0]
    nkw = ktab.shape[0]
    per_w = B // NW
    n_chunks = per_w // CH
    mesh = plsc.VectorSubcoreMesh(core_axis_name="c", subcore_axis_name="s")

    @functools.partial(
        pl.kernel,
        mesh=mesh,
        out_type=jax.ShapeDtypeStruct((NNEG, B), jnp.float32),
        compiler_params=pltpu.CompilerParams(
            needs_layout_passes=False, use_tc_tiling_on_sc=False),
        scratch_types=[
            pltpu.VMEM((CH,), jnp.int32),
            pltpu.VMEM((CH,), jnp.int32),
            pltpu.VMEM((CH,), jnp.int32),
            pltpu.VMEM((NNEG, CH), jnp.int32),
            pltpu.VMEM((CH * L_KW,), jnp.int32),
            pltpu.VMEM((L_KW, CH), jnp.int32),
            pltpu.VMEM((CH, D), jnp.float32),
            pltpu.VMEM((CH, D), jnp.float32),
            pltpu.VMEM((NNEG, CH, D), jnp.float32),
            pltpu.VMEM((CH * L_KW, D), jnp.float32),
            pltpu.VMEM((NNEG, CH), jnp.float32),
            pltpu.SemaphoreType.DMA,
        ],
    )
    def k(utab_h, itab_h, ktab_h, uid_h, iid_h, kw_h, qs_h, neg_h, out_h,
          uid_v, iid_v, qs_v, nid_v, kwraw_v, kwc_v, urows, itrows, negrows,
          kwrows, out_v, sem):
        wid = lax.axis_index("s") * NC + lax.axis_index("c")
        iota = lax.iota(jnp.int32, LANES)

        def chunk_body(c, carry):
            base = wid * per_w + c * CH
            descs = [
                pltpu.async_copy(uid_h.at[pl.ds(base, CH)], uid_v, sem),
                pltpu.async_copy(iid_h.at[pl.ds(base, CH)], iid_v, sem),
                pltpu.async_copy(qs_h.at[pl.ds(base, CH)], qs_v, sem),
                pltpu.async_copy(kw_h.at[pl.ds(base * L_KW, CH * L_KW)],
                                 kwraw_v, sem),
            ]
            for n in range(NNEG):
                descs.append(pltpu.async_copy(
                    neg_h.at[n, pl.ds(base, CH)], nid_v.at[n], sem))
            for dsc in descs:
                dsc.wait()

            def clamp_body(j, carry2):
                for t in range(CH // LANES):
                    ids = kwraw_v[pl.ds(j * CH + t * LANES, LANES)]
                    kwc_v[j, pl.ds(t * LANES, LANES)] = jnp.where(
                        ids < nkw, ids, 0)
                return carry2
            lax.fori_loop(0, L_KW, clamp_body, 0)

            gds = [
                pltpu.async_copy(utab_h.at[uid_v], urows, sem),
                pltpu.async_copy(itab_h.at[iid_v], itrows, sem),
            ]
            for n in range(NNEG):
                gds.append(pltpu.async_copy(
                    itab_h.at[nid_v.at[n]], negrows.at[n], sem))
            for j in range(L_KW):
                gds.append(pltpu.async_copy(
                    ktab_h.at[kwc_v.at[j]],
                    kwrows.at[pl.ds(j * CH, CH)], sem))
            for dsc in gds:
                dsc.wait()

            def group_body(g, carry2):
                s_loc = g * LANES + iota
                s20 = s_loc * L_KW
                zero = jnp.zeros((LANES,), jnp.float32)

                def l_body(l, acc):
                    idx = s20 + l
                    ids16 = plsc.load_gather(kwraw_v, [idx])
                    m = ids16 < nkw
                    new = []
                    for d in range(D):
                        v = plsc.load_gather(kwrows, [idx, _splat(d)])
                        new.append(acc[d] + jnp.where(m, v, 0.0))
                    return tuple(new)

                acc = lax.fori_loop(0, L_KW, l_body, (zero,) * D)
                qs16 = plsc.load_gather(qs_v, [s_loc])
                qsf = jnp.clip(qs16, 1, L_KW).astype(jnp.float32)
                inv = 1.0 / qsf
                q = [a * inv for a in acc]

                a = zero
                b = [zero] * NNEG
                for d in range(D):
                    dcol = _splat(d)
                    ud = plsc.load_gather(urows, [s_loc, dcol])
                    itd = plsc.load_gather(itrows, [s_loc, dcol])
                    sd = ud + q[d]
                    a = a + sd * itd
                    for n in range(NNEG):
                        nd = plsc.load_gather(negrows, [_splat(n), s_loc, dcol])
                        b[n] = b[n] + sd * nd
                for n in range(NNEG):
                    plsc.store_scatter(out_v, [_splat(n), s_loc], a - b[n])
                return carry2

            lax.fori_loop(0, CH // LANES, group_body, 0)

            wds = [pltpu.async_copy(out_v.at[n], out_h.at[n, pl.ds(base, CH)],
                                    sem) for n in range(NNEG)]
            for dsc in wds:
                dsc.wait()
            return carry

        lax.fori_loop(0, n_chunks, chunk_body, 0)

    return k(utab, itab, ktab, uid, iid, kw_flat, qs, negid)


def _loss_tc(diffs):
    nb = diffs.shape[0] * diffs.shape[1]

    def body(x_ref, o_ref):
        x = x_ref[...]
        sp = jnp.maximum(-x, 0.0) + jnp.log1p(jnp.exp(-jnp.abs(x)))
        o_ref[...] = jnp.sum(sp, keepdims=True) * (1.0 / nb)

    return pl.pallas_call(
        body,
        out_shape=jax.ShapeDtypeStruct((1, 1), jnp.float32),
    )(diffs)


def kernel(user_table, item_table, keyword_table, user_ids, item_ids,
           keyword_ids, query_sizes, negative_item_ids):
    uid = user_ids.astype(jnp.int32)
    iid = item_ids.astype(jnp.int32)
    kw_flat = keyword_ids.astype(jnp.int32).reshape(-1)
    qs = query_sizes.astype(jnp.int32)
    neg = negative_item_ids.astype(jnp.int32)
    diffs = _fm_diffs_sc(user_table, item_table, keyword_table,
                         uid, iid, kw_flat, qs, neg)
    return _loss_tc(diffs)[0, 0]
